# baseline (speedup 1.0000x reference)
import jax
import jax.numpy as jnp
from jax import lax
from jax.experimental import pallas as pl
from jax.experimental.pallas import tpu as pltpu

_N, _C = 32, 768
_K = _C // 2
_ONE_BITS = 0x3F800000


def _mask_body(mask_ref, W1_ref, b1_ref, W2_ref, b2_ref, m_ref, cm_ref):
    h = jnp.dot(mask_ref[:], W1_ref[:], preferred_element_type=jnp.float32)
    h = jnp.maximum(h + b1_ref[:], 0.0)
    z = jnp.dot(h, W2_ref[:], preferred_element_type=jnp.float32) + b2_ref[:]
    m = jax.nn.sigmoid(z)
    m_ref[:] = m

    bits = lax.bitcast_convert_type(m, jnp.int32)

    def step(_, carry):
        lo, hi = carry
        mid = (lo + hi) >> 1
        cnt = jnp.sum((bits <= mid).astype(jnp.int32), axis=1, keepdims=True)
        ge = cnt >= _K
        return jnp.where(ge, lo, mid), jnp.where(ge, mid, hi)

    lo0 = jnp.full((_N, 1), -1, jnp.int32)
    hi0 = jnp.full((_N, 1), _ONE_BITS, jnp.int32)
    _, kth_bits = lax.fori_loop(0, 31, step, (lo0, hi0))
    cm_ref[:] = (bits > kth_bits).astype(jnp.float32)


_SLAB_ROWS = 49
_CH_PER_SLAB = 8
_SLABS_PER_BLK = 16


def _swap_body(cm_ref, lst_ref, gui_ref, ol_ref, og_ref):
    rows = lax.broadcasted_iota(jnp.int32, (_SLAB_ROWS, 128), 0)
    cols = lax.broadcasted_iota(jnp.int32, (_SLAB_ROWS, 128), 1)
    e = rows * 128 + cols
    for s in range(_SLABS_PER_BLK):
        x = jnp.full((_SLAB_ROWS, 128), cm_ref[0, _CH_PER_SLAB * s], jnp.float32)
        for j in range(1, _CH_PER_SLAB):
            x = jnp.where(e >= 784 * j, cm_ref[0, _CH_PER_SLAB * s + j], x)
        sel = x > 0.5
        sl = pl.ds(_SLAB_ROWS * s, _SLAB_ROWS)
        l = lst_ref[sl, :]
        g = gui_ref[sl, :]
        ol_ref[sl, :] = jnp.where(sel, g, l)
        og_ref[sl, :] = jnp.where(sel, l, g)


def kernel(lst, gui, mask, W1, b1, W2, b2):
    N, C, H, W = lst.shape
    HW = H * W

    m, cm = pl.pallas_call(
        _mask_body,
        out_shape=(
            jax.ShapeDtypeStruct((N, C), jnp.float32),
            jax.ShapeDtypeStruct((N, C), jnp.float32),
        ),
    )(mask, W1, b1.reshape(1, C), W2, b2.reshape(1, C))

    total_rows = N * C * HW // 128
    blk_rows = _SLAB_ROWS * _SLABS_PER_BLK
    ch_per_blk = _CH_PER_SLAB * _SLABS_PER_BLK
    lst2 = lst.reshape(total_rows, 128)
    gui2 = gui.reshape(total_rows, 128)
    cm2 = cm.reshape(1, N * C)

    ol, og = pl.pallas_call(
        _swap_body,
        grid=(total_rows // blk_rows,),
        in_specs=[
            pl.BlockSpec((1, ch_per_blk), lambda i: (0, i),
                         memory_space=pltpu.SMEM),
            pl.BlockSpec((blk_rows, 128), lambda i: (i, 0)),
            pl.BlockSpec((blk_rows, 128), lambda i: (i, 0)),
        ],
        out_specs=(
            pl.BlockSpec((blk_rows, 128), lambda i: (i, 0)),
            pl.BlockSpec((blk_rows, 128), lambda i: (i, 0)),
        ),
        out_shape=(
            jax.ShapeDtypeStruct((total_rows, 128), jnp.float32),
            jax.ShapeDtypeStruct((total_rows, 128), jnp.float32),
        ),
        compiler_params=pltpu.CompilerParams(
            dimension_semantics=("arbitrary",),
        ),
    )(cm2, lst2, gui2)

    return ol.reshape(N, C, H, W), og.reshape(N, C, H, W), m


# speedup vs baseline: 13.1503x; 13.1503x over previous
import jax
import jax.numpy as jnp
from jax import lax
from jax.experimental import pallas as pl
from jax.experimental.pallas import tpu as pltpu

_N, _C = 32, 768
_K = _C // 2
_ONE_BITS = 0x3F800000


def _mask_body(mask_ref, W1_ref, b1_ref, W2_ref, b2_ref, m_ref, cm_ref):
    h = jnp.dot(mask_ref[:], W1_ref[:], preferred_element_type=jnp.float32)
    h = jnp.maximum(h + b1_ref[:], 0.0)
    z = jnp.dot(h, W2_ref[:], preferred_element_type=jnp.float32) + b2_ref[:]
    m = jax.nn.sigmoid(z)
    m_ref[:] = m

    bits = lax.bitcast_convert_type(m, jnp.int32)

    def step(_, carry):
        lo, hi = carry
        mid = (lo + hi) >> 1
        cnt = jnp.sum((bits <= mid).astype(jnp.int32), axis=1, keepdims=True)
        ge = cnt >= _K
        return jnp.where(ge, lo, mid), jnp.where(ge, mid, hi)

    lo0 = jnp.full((_N, 1), -1, jnp.int32)
    hi0 = jnp.full((_N, 1), _ONE_BITS, jnp.int32)
    _, kth_bits = lax.fori_loop(0, 31, step, (lo0, hi0))
    cm_ref[:] = (bits > kth_bits).astype(jnp.float32)


def _swap_body(cm_ref, lst_ref, gui_ref, ol_ref, og_ref):
    sel = (cm_ref[:] > 0.5)[None, :, :]
    l = lst_ref[:]
    g = gui_ref[:]
    ol_ref[:] = jnp.where(sel, g, l)
    og_ref[:] = jnp.where(sel, l, g)


def kernel(lst, gui, mask, W1, b1, W2, b2):
    N, C, H, W = lst.shape
    HW = H * W

    m, cm = pl.pallas_call(
        _mask_body,
        out_shape=(
            jax.ShapeDtypeStruct((N, C), jnp.float32),
            jax.ShapeDtypeStruct((N, C), jnp.float32),
        ),
    )(mask, W1, b1.reshape(1, C), W2, b2.reshape(1, C))

    B = 16
    lst2 = lst.transpose(2, 3, 0, 1).reshape(HW, N, C)
    gui2 = gui.transpose(2, 3, 0, 1).reshape(HW, N, C)

    ol, og = pl.pallas_call(
        _swap_body,
        grid=(HW // B,),
        in_specs=[
            pl.BlockSpec((N, C), lambda i: (0, 0)),
            pl.BlockSpec((B, N, C), lambda i: (i, 0, 0)),
            pl.BlockSpec((B, N, C), lambda i: (i, 0, 0)),
        ],
        out_specs=(
            pl.BlockSpec((B, N, C), lambda i: (i, 0, 0)),
            pl.BlockSpec((B, N, C), lambda i: (i, 0, 0)),
        ),
        out_shape=(
            jax.ShapeDtypeStruct((HW, N, C), jnp.float32),
            jax.ShapeDtypeStruct((HW, N, C), jnp.float32),
        ),
        compiler_params=pltpu.CompilerParams(
            dimension_semantics=("arbitrary",),
        ),
    )(cm, lst2, gui2)

    ol = ol.reshape(H, W, N, C).transpose(2, 3, 0, 1)
    og = og.reshape(H, W, N, C).transpose(2, 3, 0, 1)
    return ol, og, m


# speedup vs baseline: 13.3789x; 1.0174x over previous
import jax
import jax.numpy as jnp
from jax import lax
from jax.experimental import pallas as pl
from jax.experimental.pallas import tpu as pltpu

_N, _C = 32, 768
_K = _C // 2
_ONE_BITS = 0x3F800000
_B = 16


def _body(mask_ref, W1_ref, b1_ref, W2_ref, b2_ref, lst_ref, gui_ref,
          m_ref, ol_ref, og_ref, cm_ref):
    @pl.when(pl.program_id(0) == 0)
    def _():
        h = jnp.dot(mask_ref[:], W1_ref[:], preferred_element_type=jnp.float32)
        h = jnp.maximum(h + b1_ref[:], 0.0)
        z = jnp.dot(h, W2_ref[:], preferred_element_type=jnp.float32) + b2_ref[:]
        m = jax.nn.sigmoid(z)
        m_ref[:] = m

        bits = lax.bitcast_convert_type(m, jnp.int32)

        def step(_, carry):
            lo, hi = carry
            mid = (lo + hi) >> 1
            cnt = jnp.sum((bits <= mid).astype(jnp.int32), axis=1, keepdims=True)
            ge = cnt >= _K
            return jnp.where(ge, lo, mid), jnp.where(ge, mid, hi)

        lo0 = jnp.full((_N, 1), -1, jnp.int32)
        hi0 = jnp.full((_N, 1), _ONE_BITS, jnp.int32)
        _, kth_bits = lax.fori_loop(0, 31, step, (lo0, hi0))
        cm_ref[:] = (bits > kth_bits).astype(jnp.float32)

    sel = (cm_ref[:] > 0.5)[None, :, :]
    l = lst_ref[:]
    g = gui_ref[:]
    ol_ref[:] = jnp.where(sel, g, l)
    og_ref[:] = jnp.where(sel, l, g)


def kernel(lst, gui, mask, W1, b1, W2, b2):
    N, C, H, W = lst.shape
    HW = H * W

    lst2 = lst.transpose(2, 3, 0, 1).reshape(HW, N, C)
    gui2 = gui.transpose(2, 3, 0, 1).reshape(HW, N, C)

    const = lambda i: (0, 0)
    blk = lambda i: (i, 0, 0)
    m, ol, og = pl.pallas_call(
        _body,
        grid=(HW // _B,),
        in_specs=[
            pl.BlockSpec(mask.shape, const),
            pl.BlockSpec(W1.shape, const),
            pl.BlockSpec((1, C), const),
            pl.BlockSpec(W2.shape, const),
            pl.BlockSpec((1, C), const),
            pl.BlockSpec((_B, N, C), blk),
            pl.BlockSpec((_B, N, C), blk),
        ],
        out_specs=(
            pl.BlockSpec((N, C), const),
            pl.BlockSpec((_B, N, C), blk),
            pl.BlockSpec((_B, N, C), blk),
        ),
        out_shape=(
            jax.ShapeDtypeStruct((N, C), jnp.float32),
            jax.ShapeDtypeStruct((HW, N, C), jnp.float32),
            jax.ShapeDtypeStruct((HW, N, C), jnp.float32),
        ),
        scratch_shapes=[pltpu.VMEM((N, C), jnp.float32)],
        compiler_params=pltpu.CompilerParams(
            dimension_semantics=("arbitrary",),
        ),
    )(mask, W1, b1.reshape(1, C), W2, b2.reshape(1, C), lst2, gui2)

    ol = ol.reshape(H, W, N, C).transpose(2, 3, 0, 1)
    og = og.reshape(H, W, N, C).transpose(2, 3, 0, 1)
    return ol, og, m


# speedup vs baseline: 14.0080x; 1.0470x over previous
import jax
import jax.numpy as jnp
from jax import lax
from jax.experimental import pallas as pl
from jax.experimental.pallas import tpu as pltpu

_N, _C = 32, 768
_K = _C // 2
_ONE_BITS = 0x3F800000
_B = 28


def _body(mask_ref, W1_ref, b1_ref, W2_ref, b2_ref, lst_ref, gui_ref,
          m_ref, ol_ref, og_ref, cm_ref):
    @pl.when(pl.program_id(0) == 0)
    def _():
        h = jnp.dot(mask_ref[:], W1_ref[:], preferred_element_type=jnp.float32)
        h = jnp.maximum(h + b1_ref[:], 0.0)
        z = jnp.dot(h, W2_ref[:], preferred_element_type=jnp.float32) + b2_ref[:]
        m = jax.nn.sigmoid(z)
        m_ref[:] = m

        bits = lax.bitcast_convert_type(m, jnp.int32)

        def step(_, carry):
            lo, hi = carry
            mid = (lo + hi) >> 1
            cnt = jnp.sum((bits <= mid).astype(jnp.int32), axis=1, keepdims=True)
            ge = cnt >= _K
            return jnp.where(ge, lo, mid), jnp.where(ge, mid, hi)

        lo0 = jnp.full((_N, 1), -1, jnp.int32)
        hi0 = jnp.full((_N, 1), _ONE_BITS, jnp.int32)
        _, kth_bits = lax.fori_loop(0, 31, step, (lo0, hi0))
        cm_ref[:] = (bits > kth_bits).astype(jnp.float32)

    sel = (cm_ref[:] > 0.5)[None, :, :]
    l = lst_ref[:]
    g = gui_ref[:]
    ol_ref[:] = jnp.where(sel, g, l)
    og_ref[:] = jnp.where(sel, l, g)


def kernel(lst, gui, mask, W1, b1, W2, b2):
    N, C, H, W = lst.shape
    HW = H * W

    lst2 = lst.transpose(2, 3, 0, 1).reshape(HW, N, C)
    gui2 = gui.transpose(2, 3, 0, 1).reshape(HW, N, C)

    const = lambda i: (0, 0)
    blk = lambda i: (i, 0, 0)
    m, ol, og = pl.pallas_call(
        _body,
        grid=(HW // _B,),
        in_specs=[
            pl.BlockSpec(mask.shape, const),
            pl.BlockSpec(W1.shape, const),
            pl.BlockSpec((1, C), const),
            pl.BlockSpec(W2.shape, const),
            pl.BlockSpec((1, C), const),
            pl.BlockSpec((_B, N, C), blk),
            pl.BlockSpec((_B, N, C), blk),
        ],
        out_specs=(
            pl.BlockSpec((N, C), const),
            pl.BlockSpec((_B, N, C), blk),
            pl.BlockSpec((_B, N, C), blk),
        ),
        out_shape=(
            jax.ShapeDtypeStruct((N, C), jnp.float32),
            jax.ShapeDtypeStruct((HW, N, C), jnp.float32),
            jax.ShapeDtypeStruct((HW, N, C), jnp.float32),
        ),
        scratch_shapes=[pltpu.VMEM((N, C), jnp.float32)],
        compiler_params=pltpu.CompilerParams(
            dimension_semantics=("arbitrary",),
        ),
    )(mask, W1, b1.reshape(1, C), W2, b2.reshape(1, C), lst2, gui2)

    ol = ol.reshape(H, W, N, C).transpose(2, 3, 0, 1)
    og = og.reshape(H, W, N, C).transpose(2, 3, 0, 1)
    return ol, og, m


# speedup vs baseline: 14.1255x; 1.0084x over previous
import jax
import jax.numpy as jnp
from jax import lax
from jax.experimental import pallas as pl
from jax.experimental.pallas import tpu as pltpu

_N, _C = 32, 768
_K = _C // 2
_ONE_BITS = 0x3F800000
_B = 49


def _body(mask_ref, W1_ref, b1_ref, W2_ref, b2_ref, lst_ref, gui_ref,
          m_ref, ol_ref, og_ref, cm_ref):
    @pl.when(pl.program_id(0) == 0)
    def _():
        h = jnp.dot(mask_ref[:], W1_ref[:], preferred_element_type=jnp.float32)
        h = jnp.maximum(h + b1_ref[:], 0.0)
        z = jnp.dot(h, W2_ref[:], preferred_element_type=jnp.float32) + b2_ref[:]
        m = jax.nn.sigmoid(z)
        m_ref[:] = m

        bits = lax.bitcast_convert_type(m, jnp.int32)

        def step(_, carry):
            lo, hi = carry
            mid = (lo + hi) >> 1
            cnt = jnp.sum((bits <= mid).astype(jnp.int32), axis=1, keepdims=True)
            ge = cnt >= _K
            return jnp.where(ge, lo, mid), jnp.where(ge, mid, hi)

        lo0 = jnp.full((_N, 1), -1, jnp.int32)
        hi0 = jnp.full((_N, 1), _ONE_BITS, jnp.int32)
        _, kth_bits = lax.fori_loop(0, 31, step, (lo0, hi0))
        cm_ref[:] = (bits > kth_bits).astype(jnp.float32)

    sel = (cm_ref[:] > 0.5)[None, :, :]
    l = lst_ref[:]
    g = gui_ref[:]
    ol_ref[:] = jnp.where(sel, g, l)
    og_ref[:] = jnp.where(sel, l, g)


def kernel(lst, gui, mask, W1, b1, W2, b2):
    N, C, H, W = lst.shape
    HW = H * W

    lst2 = lst.transpose(2, 3, 0, 1).reshape(HW, N, C)
    gui2 = gui.transpose(2, 3, 0, 1).reshape(HW, N, C)

    const = lambda i: (0, 0)
    blk = lambda i: (i, 0, 0)
    m, ol, og = pl.pallas_call(
        _body,
        grid=(HW // _B,),
        in_specs=[
            pl.BlockSpec(mask.shape, const),
            pl.BlockSpec(W1.shape, const),
            pl.BlockSpec((1, C), const),
            pl.BlockSpec(W2.shape, const),
            pl.BlockSpec((1, C), const),
            pl.BlockSpec((_B, N, C), blk),
            pl.BlockSpec((_B, N, C), blk),
        ],
        out_specs=(
            pl.BlockSpec((N, C), const),
            pl.BlockSpec((_B, N, C), blk),
            pl.BlockSpec((_B, N, C), blk),
        ),
        out_shape=(
            jax.ShapeDtypeStruct((N, C), jnp.float32),
            jax.ShapeDtypeStruct((HW, N, C), jnp.float32),
            jax.ShapeDtypeStruct((HW, N, C), jnp.float32),
        ),
        scratch_shapes=[pltpu.VMEM((N, C), jnp.float32)],
        compiler_params=pltpu.CompilerParams(
            dimension_semantics=("arbitrary",),
        ),
    )(mask, W1, b1.reshape(1, C), W2, b2.reshape(1, C), lst2, gui2)

    ol = ol.reshape(H, W, N, C).transpose(2, 3, 0, 1)
    og = og.reshape(H, W, N, C).transpose(2, 3, 0, 1)
    return ol, og, m


# speedup vs baseline: 14.1646x; 1.0028x over previous
import jax
import jax.numpy as jnp
from jax import lax
from jax.experimental import pallas as pl
from jax.experimental.pallas import tpu as pltpu

_N, _C = 32, 768
_K = _C // 2
_ONE_BITS = 0x3F800000
_B = 56


def _body(mask_ref, W1_ref, b1_ref, W2_ref, b2_ref, lst_ref, gui_ref,
          m_ref, ol_ref, og_ref, cm_ref):
    @pl.when(pl.program_id(0) == 0)
    def _():
        h = jnp.dot(mask_ref[:], W1_ref[:], preferred_element_type=jnp.float32)
        h = jnp.maximum(h + b1_ref[:], 0.0)
        z = jnp.dot(h, W2_ref[:], preferred_element_type=jnp.float32) + b2_ref[:]
        m = jax.nn.sigmoid(z)
        m_ref[:] = m

        bits = lax.bitcast_convert_type(m, jnp.int32)

        def step(_, carry):
            lo, hi = carry
            mid = (lo + hi) >> 1
            cnt = jnp.sum((bits <= mid).astype(jnp.int32), axis=1, keepdims=True)
            ge = cnt >= _K
            return jnp.where(ge, lo, mid), jnp.where(ge, mid, hi)

        lo0 = jnp.full((_N, 1), -1, jnp.int32)
        hi0 = jnp.full((_N, 1), _ONE_BITS, jnp.int32)
        _, kth_bits = lax.fori_loop(0, 31, step, (lo0, hi0))
        cm_ref[:] = (bits > kth_bits).astype(jnp.float32)

    sel = (cm_ref[:] > 0.5)[None, :, :]
    l = lst_ref[:]
    g = gui_ref[:]
    ol_ref[:] = jnp.where(sel, g, l)
    og_ref[:] = jnp.where(sel, l, g)


def kernel(lst, gui, mask, W1, b1, W2, b2):
    N, C, H, W = lst.shape
    HW = H * W

    lst2 = lst.transpose(2, 3, 0, 1).reshape(HW, N, C)
    gui2 = gui.transpose(2, 3, 0, 1).reshape(HW, N, C)

    const = lambda i: (0, 0)
    blk = lambda i: (i, 0, 0)
    m, ol, og = pl.pallas_call(
        _body,
        grid=(HW // _B,),
        in_specs=[
            pl.BlockSpec(mask.shape, const),
            pl.BlockSpec(W1.shape, const),
            pl.BlockSpec((1, C), const),
            pl.BlockSpec(W2.shape, const),
            pl.BlockSpec((1, C), const),
            pl.BlockSpec((_B, N, C), blk),
            pl.BlockSpec((_B, N, C), blk),
        ],
        out_specs=(
            pl.BlockSpec((N, C), const),
            pl.BlockSpec((_B, N, C), blk),
            pl.BlockSpec((_B, N, C), blk),
        ),
        out_shape=(
            jax.ShapeDtypeStruct((N, C), jnp.float32),
            jax.ShapeDtypeStruct((HW, N, C), jnp.float32),
            jax.ShapeDtypeStruct((HW, N, C), jnp.float32),
        ),
        scratch_shapes=[pltpu.VMEM((N, C), jnp.float32)],
        compiler_params=pltpu.CompilerParams(
            dimension_semantics=("arbitrary",),
        ),
    )(mask, W1, b1.reshape(1, C), W2, b2.reshape(1, C), lst2, gui2)

    ol = ol.reshape(H, W, N, C).transpose(2, 3, 0, 1)
    og = og.reshape(H, W, N, C).transpose(2, 3, 0, 1)
    return ol, og, m
